# baseline (speedup 1.0000x reference)
import functools

import jax
import jax.numpy as jnp
from jax import lax
from jax.experimental import pallas as pl
from jax.experimental.pallas import tpu as pltpu
from jax.experimental.pallas import tpu_sc as plsc

N = 16384
C = 26
CARD = 100
D = C * CARD
NW = 32
ROWS_PER_W = N // NW
G = 16
NGROUPS = ROWS_PER_W // G
TILE = G * D

_OUT_DTYPE = jax.dtypes.canonicalize_dtype(jnp.int64)

_mesh = plsc.VectorSubcoreMesh(core_axis_name="c", subcore_axis_name="s")


@functools.partial(
    pl.kernel,
    mesh=_mesh,
    compiler_params=pltpu.CompilerParams(needs_layout_passes=False),
    out_type=jax.ShapeDtypeStruct((N * D,), _OUT_DTYPE),
    scratch_types=[
        pltpu.VMEM((ROWS_PER_W * C,), jnp.int32),
        pltpu.VMEM((TILE,), _OUT_DTYPE),
        pltpu.VMEM((TILE,), _OUT_DTYPE),
        pltpu.SemaphoreType.DMA,
        pltpu.SemaphoreType.DMA,
    ],
)
def _one_hot_sc(x_hbm, out_hbm, xv, buf0, buf1, sem0, sem1):
    wid = lax.axis_index("s") * 2 + lax.axis_index("c")
    row_base = wid * ROWS_PER_W

    pltpu.sync_copy(x_hbm.at[pl.ds(row_base * C, ROWS_PER_W * C)], xv)

    lanes = lax.iota(jnp.int32, 16)
    row_off = lanes * D
    xrow_off = lanes * C
    ones = jnp.full((16,), 1, _OUT_DTYPE)
    zeros = jnp.zeros((16,), _OUT_DTYPE)
    zvec = jnp.zeros((16,), _OUT_DTYPE)

    bufs = (buf0, buf1)
    sems = (sem0, sem1)

    def _zero(i, _):
        buf0[pl.ds(i * 16, 16)] = zvec
        buf1[pl.ds(i * 16, 16)] = zvec
        return _
    lax.fori_loop(0, TILE // 16, _zero, 0)

    def _scatter_group(buf, g, data):
        for c in range(C):
            vals = plsc.load_gather(xv, [xrow_off + (g * (G * C) + c)])
            idx = row_off + (vals + (c * CARD))
            plsc.store_scatter(buf, [idx], data)

    def _dma_out(buf, sem, g):
        dst = out_hbm.at[pl.ds((row_base + g * G) * D, TILE)]
        pltpu.async_copy(buf, dst, sem)

    def _dma_wait(buf, sem):
        pltpu.make_async_copy(buf, out_hbm.at[pl.ds(0, TILE)], sem).wait()

    for b in range(2):
        _scatter_group(bufs[b], b, ones)
        _dma_out(bufs[b], sems[b], b)

    def _body(i, _):
        for b in range(2):
            g = i * 2 + 2 + b
            _dma_wait(bufs[b], sems[b])
            _scatter_group(bufs[b], g - 2, zeros)
            _scatter_group(bufs[b], g, ones)
            _dma_out(bufs[b], sems[b], g)
        return _
    lax.fori_loop(0, (NGROUPS - 2) // 2, _body, 0)

    for b in range(2):
        _dma_wait(bufs[b], sems[b])


def kernel(x):
    flat = _one_hot_sc(jnp.asarray(x, jnp.int32).reshape(-1))
    return flat.reshape(N, D)


# speedup vs baseline: 4.4209x; 4.4209x over previous
import functools

import jax
import jax.numpy as jnp
from jax import lax
from jax.experimental import pallas as pl
from jax.experimental.pallas import tpu as pltpu
from jax.experimental.pallas import tpu_sc as plsc

N = 16384
C = 26
CARD = 100
D = C * CARD
NW = 32
SAMPLES_PER_W = N // NW
FB = 2 * CARD
SB = 256
NSUB = SB // 16
NCHUNK = C // 2 * 2

_OUT_DTYPE = jax.dtypes.canonicalize_dtype(jnp.int64)

_mesh = plsc.VectorSubcoreMesh(core_axis_name="c", subcore_axis_name="s")


@functools.partial(
    pl.kernel,
    mesh=_mesh,
    compiler_params=pltpu.CompilerParams(needs_layout_passes=False),
    out_type=jax.ShapeDtypeStruct((D, N), _OUT_DTYPE),
    scratch_types=[
        pltpu.VMEM((SAMPLES_PER_W * C,), jnp.int32),
        pltpu.VMEM((FB, SB), _OUT_DTYPE),
        pltpu.VMEM((FB, SB), _OUT_DTYPE),
        pltpu.SemaphoreType.DMA,
        pltpu.SemaphoreType.DMA,
    ],
)
def _one_hot_sc(x_hbm, out_hbm, xv, buf0, buf1, sem0, sem1):
    wid = lax.axis_index("s") * 2 + lax.axis_index("c")
    base = wid * SAMPLES_PER_W

    pltpu.sync_copy(x_hbm.at[pl.ds(base * C, SAMPLES_PER_W * C)], xv)

    lanes = lax.iota(jnp.int32, 16)
    ones = jnp.full((16,), 1, _OUT_DTYPE)
    zeros = jnp.zeros((16,), _OUT_DTYPE)
    zvec = jnp.zeros((16,), _OUT_DTYPE)

    bufs = (buf0, buf1)
    sems = (sem0, sem1)

    def _zero(r, _):
        for k in range(NSUB):
            buf0[r, pl.ds(k * 16, 16)] = zvec
            buf1[r, pl.ds(k * 16, 16)] = zvec
        return _
    lax.fori_loop(0, FB, _zero, 0)

    def _scatter_chunk(buf, c2, h, data):
        for c_off in range(2):
            for k in range(NSUB):
                s = k * 16
                xi = lanes * C + ((h * SB + s) * C + 2 * c2 + c_off)
                vals = plsc.load_gather(xv, [xi])
                feat = vals + (c_off * CARD)
                col = lanes + s
                plsc.store_scatter(buf, [feat, col], data)

    def _dma_out(buf, sem, c2, h):
        dst = out_hbm.at[pl.ds(c2 * FB, FB), pl.ds(base + h * SB, SB)]
        pltpu.async_copy(buf, dst, sem)

    def _dma_wait(buf, sem):
        dst = out_hbm.at[pl.ds(0, FB), pl.ds(base, SB)]
        pltpu.make_async_copy(buf, dst, sem).wait()

    for b in range(2):
        _scatter_chunk(bufs[b], 0, b, ones)
        _dma_out(bufs[b], sems[b], 0, b)

    def _body(i, _):
        for b in range(2):
            c2 = i + 1
            _dma_wait(bufs[b], sems[b])
            _scatter_chunk(bufs[b], c2 - 1, b, zeros)
            _scatter_chunk(bufs[b], c2, b, ones)
            _dma_out(bufs[b], sems[b], c2, b)
        return _
    lax.fori_loop(0, (NCHUNK - 2) // 2, _body, 0)

    for b in range(2):
        _dma_wait(bufs[b], sems[b])


def kernel(x):
    out_t = _one_hot_sc(jnp.asarray(x, jnp.int32).reshape(-1))
    return out_t.T


# speedup vs baseline: 4.9738x; 1.1251x over previous
import functools

import jax
import jax.numpy as jnp
from jax import lax
from jax.experimental import pallas as pl
from jax.experimental.pallas import tpu as pltpu
from jax.experimental.pallas import tpu_sc as plsc

N = 16384
C = 26
CARD = 100
D = C * CARD
NW = 32
SAMPLES_PER_W = N // NW
FB = 2 * CARD
SB = 128
NH = SAMPLES_PER_W // SB
NSUB = SB // 16
NCHUNK = (C // 2) * NH

_OUT_DTYPE = jax.dtypes.canonicalize_dtype(jnp.int64)

_mesh = plsc.VectorSubcoreMesh(core_axis_name="c", subcore_axis_name="s")


@functools.partial(
    pl.kernel,
    mesh=_mesh,
    compiler_params=pltpu.CompilerParams(needs_layout_passes=False),
    out_type=jax.ShapeDtypeStruct((D, N), _OUT_DTYPE),
    scratch_types=[
        pltpu.VMEM((SAMPLES_PER_W, C), jnp.int32),
        pltpu.VMEM((FB, SB), _OUT_DTYPE),
        pltpu.VMEM((FB, SB), _OUT_DTYPE),
        pltpu.SemaphoreType.DMA,
        pltpu.SemaphoreType.DMA,
    ],
)
def _one_hot_sc(x_hbm, out_hbm, xv, buf0, buf1, sem0, sem1):
    wid = lax.axis_index("s") * 2 + lax.axis_index("c")
    base = wid * SAMPLES_PER_W

    pltpu.sync_copy(x_hbm.at[pl.ds(base, SAMPLES_PER_W), :], xv)

    lanes = lax.iota(jnp.int32, 16)
    ones = jnp.full((16,), 1, _OUT_DTYPE)
    zeros = jnp.zeros((16,), _OUT_DTYPE)
    zvec = jnp.zeros((16,), _OUT_DTYPE)

    bufs = (buf0, buf1)
    sems = (sem0, sem1)

    def _zero(r, _):
        for k in range(NSUB):
            buf0[r, pl.ds(k * 16, 16)] = zvec
            buf1[r, pl.ds(k * 16, 16)] = zvec
        return _
    lax.fori_loop(0, FB, _zero, 0)

    def _scatter_chunk(buf, q, data):
        c2 = q // NH
        h = q - c2 * NH
        for c_off in range(2):
            c = jnp.full((16,), 2 * c2 + c_off, jnp.int32)
            for k in range(NSUB):
                s = k * 16
                row = lanes + (h * SB + s)
                vals = plsc.load_gather(xv, [row, c])
                feat = vals + (c_off * CARD)
                col = lanes + s
                plsc.store_scatter(buf, [feat, col], data)

    def _dma_out(buf, sem, q):
        c2 = q // NH
        h = q - c2 * NH
        dst = out_hbm.at[pl.ds(c2 * FB, FB), pl.ds(base + h * SB, SB)]
        pltpu.async_copy(buf, dst, sem)

    def _dma_wait(buf, sem):
        dst = out_hbm.at[pl.ds(0, FB), pl.ds(base, SB)]
        pltpu.make_async_copy(buf, dst, sem).wait()

    for b in range(2):
        _scatter_chunk(bufs[b], b, ones)
        _dma_out(bufs[b], sems[b], b)

    def _body(i, _):
        for b in range(2):
            q = i * 2 + 2 + b
            _dma_wait(bufs[b], sems[b])
            _scatter_chunk(bufs[b], q - 2, zeros)
            _scatter_chunk(bufs[b], q, ones)
            _dma_out(bufs[b], sems[b], q)
        return _
    lax.fori_loop(0, (NCHUNK - 2) // 2, _body, 0)

    for b in range(2):
        _dma_wait(bufs[b], sems[b])


def kernel(x):
    out_t = _one_hot_sc(jnp.asarray(x, jnp.int32))
    return out_t.T


# speedup vs baseline: 5.1213x; 1.0297x over previous
import functools

import jax
import jax.numpy as jnp
from jax import lax
from jax.experimental import pallas as pl
from jax.experimental.pallas import tpu as pltpu
from jax.experimental.pallas import tpu_sc as plsc

N = 16384
C = 26
CARD = 100
D = C * CARD
NW = 32
SAMPLES_PER_W = N // NW
FB = 2 * CARD
SB = 256
NH = SAMPLES_PER_W // SB
NSUB = SB // 16
NCHUNK = (C // 2) * NH

_OUT_DTYPE = jax.dtypes.canonicalize_dtype(jnp.int64)

_mesh = plsc.VectorSubcoreMesh(core_axis_name="c", subcore_axis_name="s")


@functools.partial(
    pl.kernel,
    mesh=_mesh,
    compiler_params=pltpu.CompilerParams(needs_layout_passes=False),
    out_type=jax.ShapeDtypeStruct((D, N), _OUT_DTYPE),
    scratch_types=[
        pltpu.VMEM((C, SAMPLES_PER_W), jnp.int32),
        pltpu.VMEM((FB, SB), _OUT_DTYPE),
        pltpu.VMEM((FB, SB), _OUT_DTYPE),
        pltpu.SemaphoreType.DMA,
        pltpu.SemaphoreType.DMA,
    ],
)
def _one_hot_sc(xt_hbm, out_hbm, xv, buf0, buf1, sem0, sem1):
    wid = lax.axis_index("s") * 2 + lax.axis_index("c")
    base = wid * SAMPLES_PER_W

    pltpu.sync_copy(xt_hbm.at[:, pl.ds(base, SAMPLES_PER_W)], xv)

    lanes = lax.iota(jnp.int32, 16)
    ones = jnp.full((16,), 1, _OUT_DTYPE)
    zeros = jnp.zeros((16,), _OUT_DTYPE)
    zvec = jnp.zeros((16,), _OUT_DTYPE)

    bufs = (buf0, buf1)
    sems = (sem0, sem1)

    def _zero(r, _):
        for k in range(NSUB):
            buf0[r, pl.ds(k * 16, 16)] = zvec
            buf1[r, pl.ds(k * 16, 16)] = zvec
        return _
    lax.fori_loop(0, FB, _zero, 0)

    def _scatter_chunk(buf, q, data):
        c2 = q // NH
        h = q - c2 * NH
        for c_off in range(2):
            c = jnp.full((16,), 2 * c2 + c_off, jnp.int32)
            for k in range(NSUB):
                s = k * 16
                row = lanes + (h * SB + s)
                vals = plsc.load_gather(xv, [c, row])
                feat = vals + (c_off * CARD)
                col = lanes + s
                plsc.store_scatter(buf, [feat, col], data)

    def _dma_out(buf, sem, q):
        c2 = q // NH
        h = q - c2 * NH
        dst = out_hbm.at[pl.ds(c2 * FB, FB), pl.ds(base + h * SB, SB)]
        pltpu.async_copy(buf, dst, sem)

    def _dma_wait(buf, sem):
        dst = out_hbm.at[pl.ds(0, FB), pl.ds(base, SB)]
        pltpu.make_async_copy(buf, dst, sem).wait()

    for b in range(2):
        _scatter_chunk(bufs[b], b, ones)
        _dma_out(bufs[b], sems[b], b)

    def _body(i, _):
        for b in range(2):
            q = i * 2 + 2 + b
            _dma_wait(bufs[b], sems[b])
            _scatter_chunk(bufs[b], q - 2, zeros)
            _scatter_chunk(bufs[b], q, ones)
            _dma_out(bufs[b], sems[b], q)
        return _
    lax.fori_loop(0, (NCHUNK - 2) // 2, _body, 0)

    for b in range(2):
        _dma_wait(bufs[b], sems[b])


def kernel(x):
    out_t = _one_hot_sc(jnp.asarray(x, jnp.int32).T)
    return out_t.T


# speedup vs baseline: 5.1655x; 1.0086x over previous
import functools

import jax
import jax.numpy as jnp
from jax import lax
from jax.experimental import pallas as pl
from jax.experimental.pallas import tpu as pltpu
from jax.experimental.pallas import tpu_sc as plsc

N = 16384
C = 26
CARD = 100
D = C * CARD
NW = 32
SAMPLES_PER_W = N // NW
FB = 2 * CARD
SB = 128
NH = SAMPLES_PER_W // SB
NSUB = SB // 16
NCHUNK = (C // 2) * NH
NBUF = 4

_OUT_DTYPE = jax.dtypes.canonicalize_dtype(jnp.int64)

_mesh = plsc.VectorSubcoreMesh(core_axis_name="c", subcore_axis_name="s")


@functools.partial(
    pl.kernel,
    mesh=_mesh,
    compiler_params=pltpu.CompilerParams(needs_layout_passes=False),
    out_type=jax.ShapeDtypeStruct((D, N), _OUT_DTYPE),
    scratch_types=[
        pltpu.VMEM((C, SAMPLES_PER_W), jnp.int32),
        pltpu.VMEM((FB, SB), _OUT_DTYPE),
        pltpu.VMEM((FB, SB), _OUT_DTYPE),
        pltpu.VMEM((FB, SB), _OUT_DTYPE),
        pltpu.VMEM((FB, SB), _OUT_DTYPE),
        pltpu.SemaphoreType.DMA,
        pltpu.SemaphoreType.DMA,
        pltpu.SemaphoreType.DMA,
        pltpu.SemaphoreType.DMA,
    ],
)
def _one_hot_sc(xt_hbm, out_hbm, xv, buf0, buf1, buf2, buf3,
                sem0, sem1, sem2, sem3):
    wid = lax.axis_index("s") * 2 + lax.axis_index("c")
    base = wid * SAMPLES_PER_W

    pltpu.sync_copy(xt_hbm.at[:, pl.ds(base, SAMPLES_PER_W)], xv)

    lanes = lax.iota(jnp.int32, 16)
    ones = jnp.full((16,), 1, _OUT_DTYPE)
    zeros = jnp.zeros((16,), _OUT_DTYPE)
    zvec = jnp.zeros((16,), _OUT_DTYPE)

    bufs = (buf0, buf1, buf2, buf3)
    sems = (sem0, sem1, sem2, sem3)

    def _zero(r, _):
        for k in range(NSUB):
            for buf in bufs:
                buf[r, pl.ds(k * 16, 16)] = zvec
        return _
    lax.fori_loop(0, FB, _zero, 0)

    def _scatter_chunk(buf, q, data):
        c2 = q // NH
        h = q - c2 * NH
        for c_off in range(2):
            c = jnp.full((16,), 2 * c2 + c_off, jnp.int32)
            for k in range(NSUB):
                s = k * 16
                row = lanes + (h * SB + s)
                vals = plsc.load_gather(xv, [c, row])
                feat = vals + (c_off * CARD)
                col = lanes + s
                plsc.store_scatter(buf, [feat, col], data)

    def _dma_out(buf, sem, q):
        c2 = q // NH
        h = q - c2 * NH
        dst = out_hbm.at[pl.ds(c2 * FB, FB), pl.ds(base + h * SB, SB)]
        pltpu.async_copy(buf, dst, sem)

    def _dma_wait(buf, sem):
        dst = out_hbm.at[pl.ds(0, FB), pl.ds(base, SB)]
        pltpu.make_async_copy(buf, dst, sem).wait()

    for b in range(NBUF):
        _scatter_chunk(bufs[b], b, ones)
        _dma_out(bufs[b], sems[b], b)

    def _body(i, _):
        for b in range(NBUF):
            q = i * NBUF + NBUF + b
            _dma_wait(bufs[b], sems[b])
            _scatter_chunk(bufs[b], q - NBUF, zeros)
            _scatter_chunk(bufs[b], q, ones)
            _dma_out(bufs[b], sems[b], q)
        return _
    lax.fori_loop(0, (NCHUNK - NBUF) // NBUF, _body, 0)

    for b in range(NBUF):
        _dma_wait(bufs[b], sems[b])


def kernel(x):
    out_t = _one_hot_sc(jnp.asarray(x, jnp.int32).T)
    return out_t.T


# speedup vs baseline: 5.4054x; 1.0464x over previous
import functools

import jax
import jax.numpy as jnp
from jax import lax
from jax.experimental import pallas as pl
from jax.experimental.pallas import tpu as pltpu
from jax.experimental.pallas import tpu_sc as plsc

N = 16384
C = 26
CARD = 100
D = C * CARD
NW = 32
SAMPLES_PER_W = N // NW
FB = 2 * CARD
SB = 128
NH = SAMPLES_PER_W // SB
NSUB = SB // 16
NCHUNK = (C // 2) * NH
NBUF = 4

_OUT_DTYPE = jax.dtypes.canonicalize_dtype(jnp.int64)

_mesh = plsc.VectorSubcoreMesh(core_axis_name="c", subcore_axis_name="s")


@functools.partial(
    pl.kernel,
    mesh=_mesh,
    compiler_params=pltpu.CompilerParams(needs_layout_passes=False),
    out_type=jax.ShapeDtypeStruct((D, N), _OUT_DTYPE),
    scratch_types=[
        pltpu.VMEM((C, SAMPLES_PER_W), jnp.int32),
        pltpu.VMEM((FB, SB), _OUT_DTYPE),
        pltpu.VMEM((FB, SB), _OUT_DTYPE),
        pltpu.VMEM((FB, SB), _OUT_DTYPE),
        pltpu.VMEM((FB, SB), _OUT_DTYPE),
        pltpu.SemaphoreType.DMA,
        pltpu.SemaphoreType.DMA,
        pltpu.SemaphoreType.DMA,
        pltpu.SemaphoreType.DMA,
        pltpu.SemaphoreType.DMA,
    ],
)
def _one_hot_sc(xt_hbm, out_hbm, xv, buf0, buf1, buf2, buf3,
                sem0, sem1, sem2, sem3, semx):
    wid = lax.axis_index("s") * 2 + lax.axis_index("c")
    base = wid * SAMPLES_PER_W

    pltpu.async_copy(xt_hbm.at[:, pl.ds(base, SAMPLES_PER_W)], xv, semx)

    lanes = lax.iota(jnp.int32, 16)
    ones = jnp.full((16,), 1, _OUT_DTYPE)
    zeros = jnp.zeros((16,), _OUT_DTYPE)
    zvec = jnp.zeros((16,), _OUT_DTYPE)

    bufs = (buf0, buf1, buf2, buf3)
    sems = (sem0, sem1, sem2, sem3)

    def _memset(buf):
        def _zero(r, _):
            for k in range(NSUB):
                buf[r, pl.ds(k * 16, 16)] = zvec
            return _
        lax.fori_loop(0, FB, _zero, 0)

    _memset(bufs[0])
    pltpu.make_async_copy(
        xt_hbm.at[:, pl.ds(base, SAMPLES_PER_W)], xv, semx).wait()

    def _scatter_chunk(buf, q, data):
        c2 = q // NH
        h = q - c2 * NH
        for c_off in range(2):
            c = jnp.full((16,), 2 * c2 + c_off, jnp.int32)
            for k in range(NSUB):
                s = k * 16
                row = lanes + (h * SB + s)
                vals = plsc.load_gather(xv, [c, row])
                feat = vals + (c_off * CARD)
                col = lanes + s
                plsc.store_scatter(buf, [feat, col], data)

    def _dma_out(buf, sem, q):
        c2 = q // NH
        h = q - c2 * NH
        dst = out_hbm.at[pl.ds(c2 * FB, FB), pl.ds(base + h * SB, SB)]
        pltpu.async_copy(buf, dst, sem)

    def _dma_wait(buf, sem):
        dst = out_hbm.at[pl.ds(0, FB), pl.ds(base, SB)]
        pltpu.make_async_copy(buf, dst, sem).wait()

    for b in range(NBUF):
        if b > 0:
            _memset(bufs[b])
        _scatter_chunk(bufs[b], b, ones)
        _dma_out(bufs[b], sems[b], b)

    def _body(i, _):
        for b in range(NBUF):
            q = i * NBUF + NBUF + b
            _dma_wait(bufs[b], sems[b])
            _scatter_chunk(bufs[b], q - NBUF, zeros)
            _scatter_chunk(bufs[b], q, ones)
            _dma_out(bufs[b], sems[b], q)
        return _
    lax.fori_loop(0, (NCHUNK - NBUF) // NBUF, _body, 0)

    for b in range(NBUF):
        _dma_wait(bufs[b], sems[b])


def kernel(x):
    out_t = _one_hot_sc(jnp.asarray(x, jnp.int32).T)
    return out_t.T


# speedup vs baseline: 5.4143x; 1.0016x over previous
import functools

import jax
import jax.numpy as jnp
from jax import lax
from jax.experimental import pallas as pl
from jax.experimental.pallas import tpu as pltpu
from jax.experimental.pallas import tpu_sc as plsc

N = 16384
C = 26
CARD = 100
D = C * CARD
NW = 32
SAMPLES_PER_W = N // NW
FB = 2 * CARD
SB = 128
NH = SAMPLES_PER_W // SB
NSUB = SB // 16
NCHUNK = (C // 2) * NH
NBUF = 4

_OUT_DTYPE = jax.dtypes.canonicalize_dtype(jnp.int64)

_mesh = plsc.VectorSubcoreMesh(core_axis_name="c", subcore_axis_name="s")


@functools.partial(
    pl.kernel,
    mesh=_mesh,
    compiler_params=pltpu.CompilerParams(needs_layout_passes=False),
    out_type=jax.ShapeDtypeStruct((D, N), _OUT_DTYPE),
    scratch_types=[
        pltpu.VMEM((C, SAMPLES_PER_W), jnp.int32),
        pltpu.VMEM((FB, SB), _OUT_DTYPE),
        pltpu.VMEM((FB, SB), _OUT_DTYPE),
        pltpu.VMEM((FB, SB), _OUT_DTYPE),
        pltpu.VMEM((FB, SB), _OUT_DTYPE),
        pltpu.SemaphoreType.DMA,
        pltpu.SemaphoreType.DMA,
        pltpu.SemaphoreType.DMA,
        pltpu.SemaphoreType.DMA,
        pltpu.SemaphoreType.DMA,
    ],
)
def _one_hot_sc(xt_hbm, out_hbm, xv, buf0, buf1, buf2, buf3,
                sem0, sem1, sem2, sem3, semx):
    wid = lax.axis_index("s") * 2 + lax.axis_index("c")
    base = wid * SAMPLES_PER_W

    pltpu.async_copy(xt_hbm.at[:, pl.ds(base, SAMPLES_PER_W)], xv, semx)

    lanes = lax.iota(jnp.int32, 16)
    ones = jnp.full((16,), 1, _OUT_DTYPE)
    zeros = jnp.zeros((16,), _OUT_DTYPE)
    zvec = jnp.zeros((16,), _OUT_DTYPE)

    bufs = (buf0, buf1, buf2, buf3)
    sems = (sem0, sem1, sem2, sem3)

    def _memset(buf):
        def _zero(r, _):
            for k in range(NSUB):
                buf[r, pl.ds(k * 16, 16)] = zvec
            return _
        lax.fori_loop(0, FB, _zero, 0)

    _memset(bufs[0])
    pltpu.make_async_copy(
        xt_hbm.at[:, pl.ds(base, SAMPLES_PER_W)], xv, semx).wait()

    def _scatter_chunk(buf, q, data):
        c2 = q // NH
        h = q - c2 * NH
        for c_off in range(2):
            c = 2 * c2 + c_off
            for k in range(NSUB):
                s = k * 16
                vals = xv[c, pl.ds(h * SB + s, 16)]
                feat = vals + (c_off * CARD)
                col = lanes + s
                plsc.store_scatter(buf, [feat, col], data)

    def _dma_out(buf, sem, q):
        c2 = q // NH
        h = q - c2 * NH
        dst = out_hbm.at[pl.ds(c2 * FB, FB), pl.ds(base + h * SB, SB)]
        pltpu.async_copy(buf, dst, sem)

    def _dma_wait(buf, sem):
        dst = out_hbm.at[pl.ds(0, FB), pl.ds(base, SB)]
        pltpu.make_async_copy(buf, dst, sem).wait()

    for b in range(NBUF):
        if b > 0:
            _memset(bufs[b])
        _scatter_chunk(bufs[b], b, ones)
        _dma_out(bufs[b], sems[b], b)

    def _body(i, _):
        for b in range(NBUF):
            q = i * NBUF + NBUF + b
            _dma_wait(bufs[b], sems[b])
            _scatter_chunk(bufs[b], q - NBUF, zeros)
            _scatter_chunk(bufs[b], q, ones)
            _dma_out(bufs[b], sems[b], q)
        return _
    lax.fori_loop(0, (NCHUNK - NBUF) // NBUF, _body, 0)

    for b in range(NBUF):
        _dma_wait(bufs[b], sems[b])


def kernel(x):
    out_t = _one_hot_sc(jnp.asarray(x, jnp.int32).T)
    return out_t.T
